# baseline (speedup 1.0000x reference)
import functools

import jax
import jax.numpy as jnp
from jax import lax
from jax.experimental import pallas as pl
from jax.experimental.pallas import tpu as pltpu
from jax.experimental.pallas import tpu_sc as plsc

NC = 2
NS = 16
L = 16
NW = NC * NS

CHUNK = 128


def _make_kernel(B, D):
    BPW = B // NW
    NCH = BPW // CHUNK
    G = BPW // L

    def body(uidx_hbm, iidx_hbm, uf_hbm, if_hbm, out_hbm,
             uidx_v, iidx_v, urows_v, irows_v, out_v, sem_u, sem_i):
        wid = lax.axis_index("s") * NC + lax.axis_index("c")
        pltpu.sync_copy(uidx_hbm.at[wid], uidx_v)
        pltpu.sync_copy(iidx_hbm.at[wid], iidx_v)

        @pl.loop(0, NCH)
        def _(j):
            for c in range(CHUNK // L):
                s = pl.ds(c * L, L)
                uidx_v[j, s] = uidx_v[j, s] - 1
                iidx_v[j, s] = iidx_v[j, s] - 1

        copies = []
        for j in range(NCH):
            r = pl.ds(j * CHUNK, CHUNK)
            copies.append(
                pltpu.async_copy(uf_hbm.at[uidx_v.at[j]], urows_v.at[r], sem_u))
            copies.append(
                pltpu.async_copy(if_hbm.at[iidx_v.at[j]], irows_v.at[r], sem_i))
        for c in copies:
            c.wait()

        lanes = lax.iota(jnp.int32, L)

        @pl.loop(0, G)
        def _(g):
            rows = g * L + lanes
            acc = jnp.zeros((L,), jnp.float32)
            for f in range(D):
                cols = (lanes + f) & (D - 1)
                gu = plsc.load_gather(urows_v, [rows, cols])
                gi = plsc.load_gather(irows_v, [rows, cols])
                acc = acc + gu * gi
            out_v[pl.ds(g * L, L)] = acc

        pltpu.sync_copy(out_v, out_hbm.at[pl.ds(wid * BPW, BPW)])

    return pl.kernel(
        body,
        out_type=jax.ShapeDtypeStruct((B,), jnp.float32),
        mesh=plsc.VectorSubcoreMesh(
            core_axis_name="c", subcore_axis_name="s",
            num_cores=NC, num_subcores=NS),
        compiler_params=pltpu.CompilerParams(
            needs_layout_passes=False, use_tc_tiling_on_sc=False),
        scratch_types=[
            pltpu.VMEM((BPW // CHUNK, CHUNK), jnp.int32),
            pltpu.VMEM((BPW // CHUNK, CHUNK), jnp.int32),
            pltpu.VMEM((BPW, D), jnp.float32),
            pltpu.VMEM((BPW, D), jnp.float32),
            pltpu.VMEM((BPW,), jnp.float32),
            pltpu.SemaphoreType.DMA,
            pltpu.SemaphoreType.DMA,
        ],
    )


@jax.jit
def kernel(data, user_factors, item_factors):
    B = data.shape[0]
    D = user_factors.shape[1]
    assert B % (NW * CHUNK) == 0 and D % L == 0
    users = data[:, 0].astype(jnp.int32).reshape(NW, B // (NW * CHUNK), CHUNK)
    items = data[:, 1].astype(jnp.int32).reshape(NW, B // (NW * CHUNK), CHUNK)
    return _make_kernel(B, D)(users, items, user_factors, item_factors)


# speedup vs baseline: 3.6961x; 3.6961x over previous
import functools

import jax
import jax.numpy as jnp
from jax import lax
from jax.experimental import pallas as pl
from jax.experimental.pallas import tpu as pltpu
from jax.experimental.pallas import tpu_sc as plsc

NC = 2
NS = 16
L = 16
NW = NC * NS

CH = 4
W = 128


def _make_kernel(B, D):
    BPW = B // NW
    NCHUNK = BPW // CH

    def body(uidx_hbm, iidx_hbm, uft_hbm, ift_hbm, out_hbm,
             uidx_v, iidx_v, usmem, ismem, uwin, iwin, out_v, sems):
        wid = lax.axis_index("s") * NC + lax.axis_index("c")
        pltpu.sync_copy(uidx_hbm.at[wid], uidx_v)
        pltpu.sync_copy(iidx_hbm.at[wid], iidx_v)
        @pl.loop(0, BPW // L)
        def _(g):
            uvec = uidx_v[pl.ds(g * L, L)]
            ivec = iidx_v[pl.ds(g * L, L)]
            for j in range(L):
                usmem[g * L + j] = uvec[j]
                ismem[g * L + j] = ivec[j]

        def fire(c):
            b = lax.rem(c, 2)

            @pl.loop(0, CH)
            def _(j):
                u = usmem[c * CH + j] - 1
                i = ismem[c * CH + j] - 1
                wu = pl.multiple_of(u - lax.rem(u, W), W)
                wi = pl.multiple_of(i - lax.rem(i, W), W)
                pltpu.async_copy(
                    uft_hbm.at[:, pl.ds(wu, W)], uwin.at[b, j], sems.at[b])
                pltpu.async_copy(
                    ift_hbm.at[:, pl.ds(wi, W)], iwin.at[b, j], sems.at[b])

        def shuffle(x, perm):
            return lax.gather(
                x, perm[:, None],
                dimension_numbers=lax.GatherDimensionNumbers(
                    offset_dims=(), collapsed_slice_dims=(0,),
                    start_index_map=(0,)),
                slice_sizes=(1,),
                mode=lax.GatherScatterMode.PROMISE_IN_BOUNDS)

        lanes = lax.iota(jnp.int32, L)
        lk = lanes >> 2
        fq = lanes & 3
        perm1 = lanes ^ 1
        perm2 = lanes ^ 2
        omask = fq == 0

        fire(0)

        @pl.loop(0, NCHUNK)
        def _(c):
            @pl.when(c < NCHUNK - 1)
            def _():
                fire(c + 1)

            b = lax.rem(c, 2)

            @pl.loop(0, CH)
            def _(j):
                pltpu.make_async_copy(
                    uft_hbm.at[:, pl.ds(0, W)], uwin.at[b, j], sems.at[b]
                ).wait()
                pltpu.make_async_copy(
                    ift_hbm.at[:, pl.ds(0, W)], iwin.at[b, j], sems.at[b]
                ).wait()

            b_vec = jnp.full((L,), b, jnp.int32)
            look = c * CH + lk
            u_vec = plsc.load_gather(uidx_v, [look]) - 1
            i_vec = plsc.load_gather(iidx_v, [look]) - 1
            o_u = u_vec & (W - 1)
            o_i = i_vec & (W - 1)
            acc = jnp.zeros((L,), jnp.float32)
            for q in range(D // 4):
                f_vec = q * 4 + fq
                gu = plsc.load_gather(uwin, [b_vec, lk, f_vec, o_u])
                gi = plsc.load_gather(iwin, [b_vec, lk, f_vec, o_i])
                acc = acc + gu * gi
            t = acc + shuffle(acc, perm1)
            t = t + shuffle(t, perm2)
            plsc.store_scatter(out_v, [look], t, mask=omask)

        pltpu.sync_copy(out_v, out_hbm.at[pl.ds(wid * BPW, BPW)])

    return pl.kernel(
        body,
        out_type=jax.ShapeDtypeStruct((B,), jnp.float32),
        mesh=plsc.VectorSubcoreMesh(
            core_axis_name="c", subcore_axis_name="s",
            num_cores=NC, num_subcores=NS),
        compiler_params=pltpu.CompilerParams(
            needs_layout_passes=False, use_tc_tiling_on_sc=True),
        scratch_types=[
            pltpu.VMEM((BPW,), jnp.int32),
            pltpu.VMEM((BPW,), jnp.int32),
            pltpu.SMEM((BPW,), jnp.int32),
            pltpu.SMEM((BPW,), jnp.int32),
            pltpu.VMEM((2, CH, D, W), jnp.float32),
            pltpu.VMEM((2, CH, D, W), jnp.float32),
            pltpu.VMEM((BPW,), jnp.float32),
            pltpu.SemaphoreType.DMA((2,)),
        ],
    )


@jax.jit
def kernel(data, user_factors, item_factors):
    B = data.shape[0]
    D = user_factors.shape[1]
    assert B % (NW * CH) == 0 and D % L == 0
    users = data[:, 0].astype(jnp.int32).reshape(NW, B // NW)
    items = data[:, 1].astype(jnp.int32).reshape(NW, B // NW)
    return _make_kernel(B, D)(users, items, user_factors.T, item_factors.T)


# speedup vs baseline: 3.9501x; 1.0687x over previous
import functools

import jax
import jax.numpy as jnp
from jax import lax
from jax.experimental import pallas as pl
from jax.experimental.pallas import tpu as pltpu
from jax.experimental.pallas import tpu_sc as plsc

NC = 2
NS = 16
L = 16
NW = NC * NS

CH = 4
W = 128
NBUF = 3


def _make_kernel(B, D):
    BPW = B // NW
    NCHUNK = BPW // CH

    def body(uidx_hbm, iidx_hbm, uft_hbm, ift_hbm, out_hbm,
             uidx_v, iidx_v, usmem, ismem, uwin, iwin, out_v, sems):
        wid = lax.axis_index("s") * NC + lax.axis_index("c")
        pltpu.sync_copy(uidx_hbm.at[wid], uidx_v)
        pltpu.sync_copy(iidx_hbm.at[wid], iidx_v)
        @pl.loop(0, BPW // L)
        def _(g):
            uvec = uidx_v[pl.ds(g * L, L)]
            ivec = iidx_v[pl.ds(g * L, L)]
            for j in range(L):
                usmem[g * L + j] = uvec[j]
                ismem[g * L + j] = ivec[j]

        def fire(c):
            b = lax.rem(c, NBUF)

            @pl.loop(0, CH)
            def _(j):
                u = usmem[c * CH + j] - 1
                i = ismem[c * CH + j] - 1
                wu = pl.multiple_of(u - lax.rem(u, W), W)
                wi = pl.multiple_of(i - lax.rem(i, W), W)
                pltpu.async_copy(
                    uft_hbm.at[:, pl.ds(wu, W)], uwin.at[b, j], sems.at[b])
                pltpu.async_copy(
                    ift_hbm.at[:, pl.ds(wi, W)], iwin.at[b, j], sems.at[b])

        def shuffle(x, perm):
            return lax.gather(
                x, perm[:, None],
                dimension_numbers=lax.GatherDimensionNumbers(
                    offset_dims=(), collapsed_slice_dims=(0,),
                    start_index_map=(0,)),
                slice_sizes=(1,),
                mode=lax.GatherScatterMode.PROMISE_IN_BOUNDS)

        lanes = lax.iota(jnp.int32, L)
        lk = lanes >> 2
        fq = lanes & 3
        perm1 = lanes ^ 1
        perm2 = lanes ^ 2
        omask = fq == 0

        for p0 in range(NBUF - 1):
            fire(p0)

        @pl.loop(0, NCHUNK)
        def _(c):
            @pl.when(c < NCHUNK - (NBUF - 1))
            def _():
                fire(c + NBUF - 1)

            b = lax.rem(c, NBUF)

            @pl.loop(0, CH)
            def _(j):
                pltpu.make_async_copy(
                    uft_hbm.at[:, pl.ds(0, W)], uwin.at[b, j], sems.at[b]
                ).wait()
                pltpu.make_async_copy(
                    ift_hbm.at[:, pl.ds(0, W)], iwin.at[b, j], sems.at[b]
                ).wait()

            b_vec = jnp.full((L,), b, jnp.int32)
            look = c * CH + lk
            u_vec = plsc.load_gather(uidx_v, [look]) - 1
            i_vec = plsc.load_gather(iidx_v, [look]) - 1
            o_u = u_vec & (W - 1)
            o_i = i_vec & (W - 1)
            acc = jnp.zeros((L,), jnp.float32)
            for q in range(D // 4):
                f_vec = q * 4 + fq
                gu = plsc.load_gather(uwin, [b_vec, lk, f_vec, o_u])
                gi = plsc.load_gather(iwin, [b_vec, lk, f_vec, o_i])
                acc = acc + gu * gi
            t = acc + shuffle(acc, perm1)
            t = t + shuffle(t, perm2)
            plsc.store_scatter(out_v, [look], t, mask=omask)

        pltpu.sync_copy(out_v, out_hbm.at[pl.ds(wid * BPW, BPW)])

    return pl.kernel(
        body,
        out_type=jax.ShapeDtypeStruct((B,), jnp.float32),
        mesh=plsc.VectorSubcoreMesh(
            core_axis_name="c", subcore_axis_name="s",
            num_cores=NC, num_subcores=NS),
        compiler_params=pltpu.CompilerParams(
            needs_layout_passes=False, use_tc_tiling_on_sc=True),
        scratch_types=[
            pltpu.VMEM((BPW,), jnp.int32),
            pltpu.VMEM((BPW,), jnp.int32),
            pltpu.SMEM((BPW,), jnp.int32),
            pltpu.SMEM((BPW,), jnp.int32),
            pltpu.VMEM((NBUF, CH, D, W), jnp.float32),
            pltpu.VMEM((NBUF, CH, D, W), jnp.float32),
            pltpu.VMEM((BPW,), jnp.float32),
            pltpu.SemaphoreType.DMA((NBUF,)),
        ],
    )


@jax.jit
def kernel(data, user_factors, item_factors):
    B = data.shape[0]
    D = user_factors.shape[1]
    assert B % (NW * CH) == 0 and D % L == 0
    users = data[:, 0].astype(jnp.int32).reshape(NW, B // NW)
    items = data[:, 1].astype(jnp.int32).reshape(NW, B // NW)
    return _make_kernel(B, D)(users, items, user_factors.T, item_factors.T)
